# baseline (speedup 1.0000x reference)
import functools

import jax
import jax.numpy as jnp
from jax.experimental import pallas as pl
from jax.experimental.pallas import tpu as pltpu

NUM_EXPERTS = 16
TOP_K = 2
INPUT_DIM = 2048
CHUNK = 512
NBUF = 4


def _router_body(num_tokens, x_hbm, w_ref, b_ref,
                 wout_ref, iout_ref, aux_ref, buf, acc_ref, sems):
    num_chunks = num_tokens // CHUNK

    def start(c):
        pltpu.make_async_copy(
            x_hbm.at[pl.ds(c * CHUNK, CHUNK), :],
            buf.at[c % NBUF],
            sems.at[c % NBUF],
        ).start()

    def wait(c):
        pltpu.make_async_copy(
            x_hbm.at[pl.ds(c * CHUNK, CHUNK), :],
            buf.at[c % NBUF],
            sems.at[c % NBUF],
        ).wait()

    for c in range(min(NBUF, num_chunks)):
        start(c)

    acc_ref[...] = jnp.zeros_like(acc_ref)

    for c in range(num_chunks):
        wait(c)
        logits = jnp.dot(buf[c % NBUF], w_ref[...],
                         preferred_element_type=jnp.float32) + b_ref[...]
        ids = jax.lax.broadcasted_iota(jnp.int32, logits.shape, 1)

        m1 = jnp.max(logits, axis=1, keepdims=True)
        i1 = jnp.min(jnp.where(logits == m1, ids, NUM_EXPERTS),
                     axis=1, keepdims=True)
        e = jnp.exp(logits - m1)
        s = jnp.sum(e, axis=1, keepdims=True)
        w1 = 1.0 / s

        masked = jnp.where(ids == i1, -jnp.inf, logits)
        m2 = jnp.max(masked, axis=1, keepdims=True)
        i2 = jnp.min(jnp.where(masked == m2, ids, NUM_EXPERTS),
                     axis=1, keepdims=True)
        w2 = jnp.exp(m2 - m1) / s

        sl = pl.ds(c * CHUNK, CHUNK)
        wout_ref[sl, :] = jnp.concatenate([w1, w2], axis=1)
        iout_ref[sl, :] = jnp.concatenate([i1, i2], axis=1)

        probs = e / s
        acc_ref[0:1, :] += jnp.sum((ids == i1).astype(jnp.float32)
                                   + (ids == i2).astype(jnp.float32),
                                   axis=0, keepdims=True)
        acc_ref[1:2, :] += jnp.sum(probs, axis=0, keepdims=True)

        if c + NBUF < num_chunks:
            start(c + NBUF)

    inv_n2 = 1.0 / (float(num_tokens) * float(num_tokens))
    aux_ref[...] = (NUM_EXPERTS * inv_n2
                    * jnp.sum(acc_ref[0:1, :] * acc_ref[1:2, :],
                              keepdims=True))


def kernel(x, W, b):
    num_tokens = x.shape[0] * x.shape[1]
    x_flat = x.reshape(num_tokens, INPUT_DIM)
    b2 = b.reshape(1, NUM_EXPERTS)

    body = functools.partial(_router_body, num_tokens)
    weights, indices, aux = pl.pallas_call(
        body,
        in_specs=[
            pl.BlockSpec(memory_space=pl.ANY),
            pl.BlockSpec(memory_space=pltpu.MemorySpace.VMEM),
            pl.BlockSpec(memory_space=pltpu.MemorySpace.VMEM),
        ],
        out_specs=[
            pl.BlockSpec(memory_space=pltpu.MemorySpace.VMEM),
            pl.BlockSpec(memory_space=pltpu.MemorySpace.VMEM),
            pl.BlockSpec(memory_space=pltpu.MemorySpace.VMEM),
        ],
        out_shape=[
            jax.ShapeDtypeStruct((num_tokens, TOP_K), jnp.float32),
            jax.ShapeDtypeStruct((num_tokens, TOP_K), jnp.int32),
            jax.ShapeDtypeStruct((1, 1), jnp.float32),
        ],
        scratch_shapes=[
            pltpu.VMEM((NBUF, CHUNK, INPUT_DIM), jnp.float32),
            pltpu.VMEM((8, NUM_EXPERTS), jnp.float32),
            pltpu.SemaphoreType.DMA((NBUF,)),
        ],
    )(x_flat, W, b2)
    return weights, indices, aux[0, 0]


# speedup vs baseline: 1.2300x; 1.2300x over previous
import functools

import jax
import jax.numpy as jnp
from jax import lax
from jax.experimental import pallas as pl
from jax.experimental.pallas import tpu as pltpu

NUM_EXPERTS = 16
TOP_K = 2
INPUT_DIM = 2048
CHUNK = 512
NBUF = 8


def _router_body(num_tokens, x_hbm, w_ref, b_ref,
                 wout_ref, iout_ref, aux_ref, buf, acc_ref, sems):
    num_chunks = num_tokens // CHUNK

    def start(c, ib):
        pltpu.make_async_copy(
            x_hbm.at[pl.ds(c * CHUNK, CHUNK), :],
            buf.at[ib], sems.at[ib]).start()

    def wait(c, ib):
        pltpu.make_async_copy(
            x_hbm.at[pl.ds(c * CHUNK, CHUNK), :],
            buf.at[ib], sems.at[ib]).wait()

    for c in range(NBUF):
        start(c, c)

    acc_ref[...] = jnp.zeros_like(acc_ref)

    def step(c, carry):
        ib = lax.rem(c, NBUF)
        wait(c, ib)
        logits = jnp.dot(buf[ib], w_ref[...],
                         preferred_element_type=jnp.float32) + b_ref[...]
        ids = jax.lax.broadcasted_iota(jnp.int32, logits.shape, 1)

        m1 = jnp.max(logits, axis=1, keepdims=True)
        i1 = jnp.min(jnp.where(logits == m1, ids, NUM_EXPERTS),
                     axis=1, keepdims=True)
        e = jnp.exp(logits - m1)
        s = jnp.sum(e, axis=1, keepdims=True)
        w1 = 1.0 / s

        masked = jnp.where(ids == i1, -jnp.inf, logits)
        m2 = jnp.max(masked, axis=1, keepdims=True)
        i2 = jnp.min(jnp.where(masked == m2, ids, NUM_EXPERTS),
                     axis=1, keepdims=True)
        w2 = jnp.exp(m2 - m1) / s

        sl = pl.ds(c * CHUNK, CHUNK)
        wout_ref[sl, :] = jnp.concatenate([w1, w2], axis=1)
        iout_ref[sl, :] = jnp.concatenate([i1, i2], axis=1)

        probs = e / s
        acc_ref[0:1, :] += jnp.sum((ids == i1).astype(jnp.float32)
                                   + (ids == i2).astype(jnp.float32),
                                   axis=0, keepdims=True)
        acc_ref[1:2, :] += jnp.sum(probs, axis=0, keepdims=True)

        @pl.when(c + NBUF < num_chunks)
        def _():
            start(c + NBUF, ib)

        return carry

    lax.fori_loop(0, num_chunks, step, 0)

    inv_n2 = 1.0 / (float(num_tokens) * float(num_tokens))
    aux_ref[...] = (NUM_EXPERTS * inv_n2
                    * jnp.sum(acc_ref[0:1, :] * acc_ref[1:2, :],
                              keepdims=True))


def kernel(x, W, b):
    num_tokens = x.shape[0] * x.shape[1]
    x_flat = x.reshape(num_tokens, INPUT_DIM)
    b2 = b.reshape(1, NUM_EXPERTS)

    body = functools.partial(_router_body, num_tokens)
    weights, indices, aux = pl.pallas_call(
        body,
        in_specs=[
            pl.BlockSpec(memory_space=pl.ANY),
            pl.BlockSpec(memory_space=pltpu.MemorySpace.VMEM),
            pl.BlockSpec(memory_space=pltpu.MemorySpace.VMEM),
        ],
        out_specs=[
            pl.BlockSpec(memory_space=pltpu.MemorySpace.VMEM),
            pl.BlockSpec(memory_space=pltpu.MemorySpace.VMEM),
            pl.BlockSpec(memory_space=pltpu.MemorySpace.VMEM),
        ],
        out_shape=[
            jax.ShapeDtypeStruct((num_tokens, TOP_K), jnp.float32),
            jax.ShapeDtypeStruct((num_tokens, TOP_K), jnp.int32),
            jax.ShapeDtypeStruct((1, 1), jnp.float32),
        ],
        scratch_shapes=[
            pltpu.VMEM((NBUF, CHUNK, INPUT_DIM), jnp.float32),
            pltpu.VMEM((8, NUM_EXPERTS), jnp.float32),
            pltpu.SemaphoreType.DMA((NBUF,)),
        ],
    )(x_flat, W, b2)
    return weights, indices, aux[0, 0]
